# speedup vs baseline: 10.4565x; 10.4565x over previous
import functools

import jax
import jax.numpy as jnp
from jax import lax
from jax.experimental import pallas as pl
from jax.experimental.pallas import tpu as pltpu
from jax.experimental.pallas import tpu_sc as plsc

B, S, H = 4, 8192, 1024
M = 1024
NC, NS = 2, 16
NW = NC * NS
RPW = (B * M) // NW
CH = 64
NCH = RPW // CH


def _make_kernel():
  mesh = plsc.VectorSubcoreMesh(core_axis_name="c", subcore_axis_name="s")

  @functools.partial(
      pl.kernel,
      mesh=mesh,
      out_type=jax.ShapeDtypeStruct((B * M, H), jnp.float32),
      scratch_types=[
          pltpu.VMEM((CH,), jnp.int32),
          pltpu.VMEM((CH, H), jnp.float32),
          pltpu.SemaphoreType.DMA,
      ],
  )
  def gather_kernel(mp_hbm, x_hbm, out_hbm, idx_v, rows_v, sem):
    wid = lax.axis_index("s") * NC + lax.axis_index("c")
    base = wid * RPW
    boff = (base // M) * S
    for c in range(NCH):
      pltpu.sync_copy(mp_hbm.at[pl.ds(base + c * CH, CH)], idx_v)
      for i in range(CH // 16):
        idx_v[pl.ds(i * 16, 16)] = idx_v[pl.ds(i * 16, 16)] + boff
      pltpu.async_copy(x_hbm.at[idx_v], rows_v, sem).wait()
      pltpu.sync_copy(rows_v, out_hbm.at[pl.ds(base + c * CH, CH)])

  return gather_kernel


_gather = _make_kernel()


@jax.jit
def kernel(x, masked_position):
  mp = masked_position.astype(jnp.int32).reshape(-1)
  xf = x.reshape(B * S, H)
  out = _gather(mp, xf)
  return out.reshape(B, M, H)
